# baseline (speedup 1.0000x reference)
import functools

import jax
import jax.numpy as jnp
from jax import lax
from jax.experimental import pallas as pl
from jax.experimental.pallas import tpu as pltpu
from jax.experimental.pallas import tpu_sc as plsc

NEG_INF = float("-inf")

_NC = 2
_NS = 16
_NW = _NC * _NS
_L = 16

_CAP = 128


def _score_body(K, R, G, CB, base, m_rows_ref, m_all_ref, w_ref, b_ref,
                scores_ref, thr_ref, gm_ref):
    i = pl.program_id(0)
    j = pl.program_id(1)
    jmax = (base + (i + 1) * R - 1) // CB

    @pl.when(j <= jmax)
    def _():
        prec = lax.Precision.DEFAULT
        bilin = lax.dot_general(
            m_rows_ref[...], w_ref[...], (((1,), (1,)), ((), ())),
            preferred_element_type=jnp.float32, precision=prec) + b_ref[...]
        scores = lax.dot_general(
            bilin, m_all_ref[...], (((1,), (1,)), ((), ())),
            preferred_element_type=jnp.float32, precision=prec)
        col = lax.broadcasted_iota(jnp.int32, (R, CB), 1) + j * CB
        row = lax.broadcasted_iota(jnp.int32, (R, CB), 0) + (base + i * R)
        scores = jnp.where(col < row, scores, NEG_INF)
        scores_ref[...] = scores
        part = scores[:, 0:G]
        for s in range(1, CB // G):
            part = jnp.maximum(part, scores[:, s * G:(s + 1) * G])

        @pl.when(j == 0)
        def _():
            gm_ref[...] = part

        @pl.when(j > 0)
        def _():
            gm_ref[...] = jnp.maximum(gm_ref[...], part)

        @pl.when(j == jmax)
        def _():
            giota = lax.broadcasted_iota(jnp.int32, (R, G), 1)

            def tb(_, carry):
                gm, _ = carry
                m = jnp.max(gm, axis=1, keepdims=True)
                idx = jnp.min(jnp.where(gm == m, giota, jnp.int32(G)),
                              axis=1, keepdims=True)
                return jnp.where(giota == idx, NEG_INF, gm), m

            _, t = lax.fori_loop(0, K, tb,
                                 (gm_ref[...], jnp.full((R, 1), NEG_INF,
                                                        jnp.float32)))
            thr_ref[...] = jnp.broadcast_to(t, (R, _L))


def _sc_body(CR, base, scores_hbm, thr_hbm, candv_hbm, candc_hbm,
             rows_v, thrs_v, ov, oi, sem0, sem1, osem0, osem1):
    wid = lax.axis_index("s") * _NC + lax.axis_index("c")
    lanes = lax.iota(jnp.int32, _L)
    T = CR // _NW
    sems = (sem0, sem1)
    osems = (osem0, osem1)

    def issue(t, slot):
        lr = wid + t * _NW
        pltpu.async_copy(scores_hbm.at[lr], rows_v.at[slot], sems[slot])
        pltpu.async_copy(thr_hbm.at[lr], thrs_v.at[slot], sems[slot])

    def drain_out(slot, lr):
        pltpu.make_async_copy(ov.at[slot], candv_hbm.at[lr],
                              osems[slot]).wait()
        pltpu.make_async_copy(oi.at[slot], candc_hbm.at[lr],
                              osems[slot]).wait()

    def process(t, slot):
        lr = wid + t * _NW
        r = base + lr
        pltpu.make_async_copy(scores_hbm.at[lr], rows_v.at[slot],
                              sems[slot]).wait()
        pltpu.make_async_copy(thr_hbm.at[lr], thrs_v.at[slot],
                              sems[slot]).wait()
        @pl.when(t >= 2)
        def _():
            drain_out(slot, lr)

        tvec = thrs_v[slot]
        rvec = jnp.full((_L,), r, jnp.int32)
        for q in range(_CAP // _L):
            ov[slot, pl.ds(q * _L, _L)] = jnp.full((_L,), NEG_INF,
                                                   jnp.float32)
            oi[slot, pl.ds(q * _L, _L)] = jnp.zeros((_L,), jnp.int32)
        nfull = r // _L
        ng = nfull // 4

        def grp(g, ptr):
            j0 = g * 4
            vs, ms, cs = [], [], []
            tot = jnp.zeros((_L,), jnp.int32)
            for u in range(4):
                v = rows_v[slot, pl.ds((j0 + u) * _L, _L)]
                m = v >= tvec
                c = plsc.all_reduce_population_count(m)
                vs.append(v)
                ms.append(m)
                cs.append(c)
                tot = tot + c

            def slow(p):
                for u in range(4):
                    def app(q, u=u):
                        s = plsc.cumsum(ms[u].astype(jnp.int32))
                        pos = jnp.minimum(q + s - 1, _CAP - 1)
                        plsc.store_scatter(ov.at[slot], [pos], vs[u],
                                           mask=ms[u])
                        plsc.store_scatter(oi.at[slot], [pos],
                                           lanes + (j0 + u) * _L,
                                           mask=ms[u])
                        return q + cs[u]

                    p = lax.cond(cs[u][0] > 0, app, lambda q: q, p)
                return p

            return lax.cond(tot[0] > 0, slow, lambda q: q, ptr)

        ptr = lax.fori_loop(0, ng, grp, jnp.zeros((_L,), jnp.int32))
        nv = (r + _L - 1) // _L

        def rem(j, ptr):
            v = rows_v[slot, pl.ds(j * _L, _L)]
            colv = lanes + j * _L
            m = (v >= tvec) & (colv < rvec)
            s = plsc.cumsum(m.astype(jnp.int32))
            pos = jnp.minimum(ptr + s - 1, _CAP - 1)
            plsc.store_scatter(ov.at[slot], [pos], v, mask=m)
            plsc.store_scatter(oi.at[slot], [pos], colv, mask=m)
            return ptr + plsc.all_reduce_population_count(m)

        lax.fori_loop(ng * 4, nv, rem, ptr)
        @pl.when(t + 2 < T)
        def _():
            issue(t + 2, slot)

        pltpu.async_copy(ov.at[slot], candv_hbm.at[lr], osems[slot])
        pltpu.async_copy(oi.at[slot], candc_hbm.at[lr], osems[slot])

    issue(0, 0)
    issue(1, 1)

    def pair(h, _):
        process(2 * h, 0)
        process(2 * h + 1, 1)
        return 0

    lax.fori_loop(0, T // 2, pair, 0)
    drain_out(0, wid + (T - 2) * _NW)
    drain_out(1, wid + (T - 1) * _NW)


def _sel_body(K, R, KP, candv_ref, candc_ref, out_v_ref, out_i_ref):
    blk = pl.program_id(0)
    candv = candv_ref[...]
    candc = candc_ref[...]
    siota = lax.broadcasted_iota(jnp.int32, (R, _CAP), 1)
    kio = lax.broadcasted_iota(jnp.int32, (R, KP), 1)
    cnt = jnp.sum((candv > NEG_INF).astype(jnp.int32), axis=1,
                  keepdims=True)

    def body(k, carry):
        candv, topv, topc = carry
        m = jnp.max(candv, axis=1, keepdims=True)
        eq = candv == m
        slot = jnp.min(jnp.where(eq, siota, jnp.int32(_CAP)), axis=1,
                       keepdims=True)
        hit = siota == slot
        colv = jnp.min(jnp.where(hit, candc, jnp.int32(0x7FFFFFFF)),
                       axis=1, keepdims=True)
        candv = jnp.where(hit, NEG_INF, candv)
        sel = kio == k
        topv = jnp.where(sel, m, topv)
        topc = jnp.where(sel, colv, topc)
        return candv, topv, topc

    _, topv, topc = lax.fori_loop(
        0, K, body,
        (candv, jnp.full((R, KP), NEG_INF, jnp.float32),
         jnp.zeros((R, KP), jnp.int32)))
    rvec = lax.broadcasted_iota(jnp.int32, (R, KP), 0) + blk * R
    tail = kio >= cnt
    outv = jnp.where(tail, NEG_INF, topv)
    outi = jnp.where(tail, rvec + kio - cnt, topc)
    out_v_ref[...] = outv[:, :K]
    out_i_ref[...] = outi[:, :K]


def kernel(mentions, W, b):
    n, f = mentions.shape
    K = min(50, n)
    R = min(256, n)
    G = min(256, n)
    assert n % R == 0 and n % G == 0 and n % _NW == 0
    KP = ((K + _L - 1) // _L) * _L
    NCH = 4 if n % (4 * 2 * _NW) == 0 else 1
    CR = n // NCH
    b2 = b.reshape(1, f)

    CB = min(2048, n)

    def stage1(base):
        def jcl(jj, ii):
            return jnp.minimum(jj, (base + (ii + 1) * R - 1) // CB)

        return pl.pallas_call(
            functools.partial(_score_body, K, R, G, CB, base),
            grid=(CR // R, n // CB),
            in_specs=[
                pl.BlockSpec((R, f), lambda i, j: (i, 0)),
                pl.BlockSpec((CB, f), lambda i, j: (jcl(j, i), 0)),
                pl.BlockSpec((f, f), lambda i, j: (0, 0)),
                pl.BlockSpec((1, f), lambda i, j: (0, 0)),
            ],
            out_specs=[
                pl.BlockSpec((R, CB), lambda i, j: (i, jcl(j, i))),
                pl.BlockSpec((R, _L), lambda i, j: (i, 0)),
            ],
            out_shape=[
                jax.ShapeDtypeStruct((CR, n), jnp.float32),
                jax.ShapeDtypeStruct((CR, _L), jnp.float32),
            ],
            scratch_shapes=[pltpu.VMEM((R, G), jnp.float32)],
        )(lax.dynamic_slice_in_dim(mentions, base, CR, 0), mentions, W, b2)

    def sc_chunk(base):
        return pl.kernel(
            functools.partial(_sc_body, CR, base),
            out_type=[
                jax.ShapeDtypeStruct((CR, _CAP), jnp.float32),
                jax.ShapeDtypeStruct((CR, _CAP), jnp.int32),
            ],
            mesh=plsc.VectorSubcoreMesh(core_axis_name="c",
                                        subcore_axis_name="s",
                                        num_cores=_NC, num_subcores=_NS),
            compiler_params=pltpu.CompilerParams(
                needs_layout_passes=False),
            scratch_types=[
                pltpu.VMEM((2, n), jnp.float32),
                pltpu.VMEM((2, _L), jnp.float32),
                pltpu.VMEM((2, _CAP), jnp.float32),
                pltpu.VMEM((2, _CAP), jnp.int32),
                pltpu.SemaphoreType.DMA,
                pltpu.SemaphoreType.DMA,
                pltpu.SemaphoreType.DMA,
                pltpu.SemaphoreType.DMA,
            ],
        )

    parts = {}
    for c in reversed(range(NCH)):
        base = c * CR
        scores_c, thr_c = stage1(base)
        parts[c] = sc_chunk(base)(scores_c, thr_c)
    candv = jnp.concatenate([parts[c][0] for c in range(NCH)], axis=0)
    candc = jnp.concatenate([parts[c][1] for c in range(NCH)], axis=0)

    R3 = min(512, n)
    out_v, out_i = pl.pallas_call(
        functools.partial(_sel_body, K, R3, KP),
        grid=(n // R3,),
        in_specs=[
            pl.BlockSpec((R3, _CAP), lambda i: (i, 0)),
            pl.BlockSpec((R3, _CAP), lambda i: (i, 0)),
        ],
        out_specs=[
            pl.BlockSpec((R3, K), lambda i: (i, 0)),
            pl.BlockSpec((R3, K), lambda i: (i, 0)),
        ],
        out_shape=[
            jax.ShapeDtypeStruct((n, K), jnp.float32),
            jax.ShapeDtypeStruct((n, K), jnp.int32),
        ],
    )(candv, candc)
    return out_v, out_i


# speedup vs baseline: 1.3339x; 1.3339x over previous
import functools

import jax
import jax.numpy as jnp
from jax import lax
from jax.experimental import pallas as pl
from jax.experimental.pallas import tpu as pltpu
from jax.experimental.pallas import tpu_sc as plsc

NEG_INF = float("-inf")

_NC = 2
_NS = 16
_NW = _NC * _NS
_L = 16

_CAP = 128


def _score_body(K, R, N, G, base, m_rows_ref, m_all_ref, w_ref, b_ref,
                scores_ref, thr_ref):
    blk = pl.program_id(0)
    prec = lax.Precision.DEFAULT
    bilin = lax.dot_general(
        m_rows_ref[...], w_ref[...], (((1,), (1,)), ((), ())),
        preferred_element_type=jnp.float32, precision=prec) + b_ref[...]
    scores = lax.dot_general(
        bilin, m_all_ref[...], (((1,), (1,)), ((), ())),
        preferred_element_type=jnp.float32, precision=prec)
    col = lax.broadcasted_iota(jnp.int32, (R, N), 1)
    row = lax.broadcasted_iota(jnp.int32, (R, N), 0) + (base + blk * R)
    scores = jnp.where(col < row, scores, NEG_INF)
    scores_ref[...] = scores
    gm = scores[:, 0:G]
    for s in range(1, N // G):
        gm = jnp.maximum(gm, scores[:, s * G:(s + 1) * G])
    giota = lax.broadcasted_iota(jnp.int32, (R, G), 1)

    def tb(_, carry):
        gm, _ = carry
        m = jnp.max(gm, axis=1, keepdims=True)
        idx = jnp.min(jnp.where(gm == m, giota, jnp.int32(G)), axis=1,
                      keepdims=True)
        return jnp.where(giota == idx, NEG_INF, gm), m

    _, t = lax.fori_loop(0, K, tb, (gm, jnp.full((R, 1), NEG_INF,
                                                 jnp.float32)))
    thr_ref[...] = jnp.broadcast_to(t, (R, _L))


def _sc_body(CR, base, scores_hbm, thr_hbm, candv_hbm, candc_hbm,
             rows_v, thrs_v, ov, oi, sem0, sem1, osem0, osem1):
    wid = lax.axis_index("s") * _NC + lax.axis_index("c")
    lanes = lax.iota(jnp.int32, _L)
    T = CR // _NW
    sems = (sem0, sem1)
    osems = (osem0, osem1)

    def issue(t, slot):
        lr = wid + t * _NW
        pltpu.async_copy(scores_hbm.at[lr], rows_v.at[slot], sems[slot])
        pltpu.async_copy(thr_hbm.at[lr], thrs_v.at[slot], sems[slot])

    def drain_out(slot, lr):
        pltpu.make_async_copy(ov.at[slot], candv_hbm.at[lr],
                              osems[slot]).wait()
        pltpu.make_async_copy(oi.at[slot], candc_hbm.at[lr],
                              osems[slot]).wait()

    def process(t, slot):
        lr = wid + t * _NW
        r = base + lr
        pltpu.make_async_copy(scores_hbm.at[lr], rows_v.at[slot],
                              sems[slot]).wait()
        pltpu.make_async_copy(thr_hbm.at[lr], thrs_v.at[slot],
                              sems[slot]).wait()
        @pl.when(t >= 2)
        def _():
            drain_out(slot, lr)

        tvec = thrs_v[slot]
        rvec = jnp.full((_L,), r, jnp.int32)
        for q in range(_CAP // _L):
            ov[slot, pl.ds(q * _L, _L)] = jnp.full((_L,), NEG_INF,
                                                   jnp.float32)
            oi[slot, pl.ds(q * _L, _L)] = jnp.zeros((_L,), jnp.int32)
        nfull = r // _L
        ng = nfull // 4

        def grp(g, ptr):
            j0 = g * 4
            vs, ms, cs = [], [], []
            tot = jnp.zeros((_L,), jnp.int32)
            for u in range(4):
                v = rows_v[slot, pl.ds((j0 + u) * _L, _L)]
                m = v >= tvec
                c = plsc.all_reduce_population_count(m)
                vs.append(v)
                ms.append(m)
                cs.append(c)
                tot = tot + c

            def slow(p):
                for u in range(4):
                    def app(q, u=u):
                        s = plsc.cumsum(ms[u].astype(jnp.int32))
                        pos = jnp.minimum(q + s - 1, _CAP - 1)
                        plsc.store_scatter(ov.at[slot], [pos], vs[u],
                                           mask=ms[u])
                        plsc.store_scatter(oi.at[slot], [pos],
                                           lanes + (j0 + u) * _L,
                                           mask=ms[u])
                        return q + cs[u]

                    p = lax.cond(cs[u][0] > 0, app, lambda q: q, p)
                return p

            return lax.cond(tot[0] > 0, slow, lambda q: q, ptr)

        ptr = lax.fori_loop(0, ng, grp, jnp.zeros((_L,), jnp.int32))
        nv = (r + _L - 1) // _L

        def rem(j, ptr):
            v = rows_v[slot, pl.ds(j * _L, _L)]
            colv = lanes + j * _L
            m = (v >= tvec) & (colv < rvec)
            s = plsc.cumsum(m.astype(jnp.int32))
            pos = jnp.minimum(ptr + s - 1, _CAP - 1)
            plsc.store_scatter(ov.at[slot], [pos], v, mask=m)
            plsc.store_scatter(oi.at[slot], [pos], colv, mask=m)
            return ptr + plsc.all_reduce_population_count(m)

        lax.fori_loop(ng * 4, nv, rem, ptr)
        @pl.when(t + 2 < T)
        def _():
            issue(t + 2, slot)

        pltpu.async_copy(ov.at[slot], candv_hbm.at[lr], osems[slot])
        pltpu.async_copy(oi.at[slot], candc_hbm.at[lr], osems[slot])

    issue(0, 0)
    issue(1, 1)

    def pair(h, _):
        process(2 * h, 0)
        process(2 * h + 1, 1)
        return 0

    lax.fori_loop(0, T // 2, pair, 0)
    drain_out(0, wid + (T - 2) * _NW)
    drain_out(1, wid + (T - 1) * _NW)


def _sel_body(K, R, KP, base, candv_ref, candc_ref, out_v_ref, out_i_ref):
    blk = pl.program_id(0)
    candv = candv_ref[...]
    candc = candc_ref[...]
    siota = lax.broadcasted_iota(jnp.int32, (R, _CAP), 1)
    kio = lax.broadcasted_iota(jnp.int32, (R, KP), 1)
    cnt = jnp.sum((candv > NEG_INF).astype(jnp.int32), axis=1,
                  keepdims=True)

    def body(k, carry):
        candv, topv, topc = carry
        m = jnp.max(candv, axis=1, keepdims=True)
        eq = candv == m
        slot = jnp.min(jnp.where(eq, siota, jnp.int32(_CAP)), axis=1,
                       keepdims=True)
        hit = siota == slot
        colv = jnp.min(jnp.where(hit, candc, jnp.int32(0x7FFFFFFF)),
                       axis=1, keepdims=True)
        candv = jnp.where(hit, NEG_INF, candv)
        sel = kio == k
        topv = jnp.where(sel, m, topv)
        topc = jnp.where(sel, colv, topc)
        return candv, topv, topc

    _, topv, topc = lax.fori_loop(
        0, K, body,
        (candv, jnp.full((R, KP), NEG_INF, jnp.float32),
         jnp.zeros((R, KP), jnp.int32)))
    rvec = lax.broadcasted_iota(jnp.int32, (R, KP), 0) + (base + blk * R)
    tail = kio >= cnt
    outv = jnp.where(tail, NEG_INF, topv)
    outi = jnp.where(tail, rvec + kio - cnt, topc)
    out_v_ref[...] = outv[:, :K]
    out_i_ref[...] = outi[:, :K]


def kernel(mentions, W, b):
    n, f = mentions.shape
    K = min(50, n)
    R = min(256, n)
    G = min(256, n)
    assert n % R == 0 and n % G == 0 and n % _NW == 0
    KP = ((K + _L - 1) // _L) * _L
    NCH = 8 if n % (8 * 2 * _NW) == 0 else 1
    CR = n // NCH
    b2 = b.reshape(1, f)

    def stage1(base):
        boff = base // R
        return pl.pallas_call(
            functools.partial(_score_body, K, R, n, G, base),
            grid=(CR // R,),
            in_specs=[
                pl.BlockSpec((R, f), lambda i: (boff + i, 0)),
                pl.BlockSpec((n, f), lambda i: (0, 0)),
                pl.BlockSpec((f, f), lambda i: (0, 0)),
                pl.BlockSpec((1, f), lambda i: (0, 0)),
            ],
            out_specs=[
                pl.BlockSpec((R, n), lambda i: (i, 0)),
                pl.BlockSpec((R, _L), lambda i: (i, 0)),
            ],
            out_shape=[
                jax.ShapeDtypeStruct((CR, n), jnp.float32),
                jax.ShapeDtypeStruct((CR, _L), jnp.float32),
            ],
        )(mentions, mentions, W, b2)

    def sc_chunk(base):
        return pl.kernel(
            functools.partial(_sc_body, CR, base),
            out_type=[
                jax.ShapeDtypeStruct((CR, _CAP), jnp.float32),
                jax.ShapeDtypeStruct((CR, _CAP), jnp.int32),
            ],
            mesh=plsc.VectorSubcoreMesh(core_axis_name="c",
                                        subcore_axis_name="s",
                                        num_cores=_NC, num_subcores=_NS),
            compiler_params=pltpu.CompilerParams(
                needs_layout_passes=False),
            scratch_types=[
                pltpu.VMEM((2, n), jnp.float32),
                pltpu.VMEM((2, _L), jnp.float32),
                pltpu.VMEM((2, _CAP), jnp.float32),
                pltpu.VMEM((2, _CAP), jnp.int32),
                pltpu.SemaphoreType.DMA,
                pltpu.SemaphoreType.DMA,
                pltpu.SemaphoreType.DMA,
                pltpu.SemaphoreType.DMA,
            ],
        )

    R3 = min(512, CR)

    def stage3(base, candv, candc):
        return pl.pallas_call(
            functools.partial(_sel_body, K, R3, KP, base),
            grid=(CR // R3,),
            in_specs=[
                pl.BlockSpec((R3, _CAP), lambda i: (i, 0)),
                pl.BlockSpec((R3, _CAP), lambda i: (i, 0)),
            ],
            out_specs=[
                pl.BlockSpec((R3, K), lambda i: (i, 0)),
                pl.BlockSpec((R3, K), lambda i: (i, 0)),
            ],
            out_shape=[
                jax.ShapeDtypeStruct((CR, K), jnp.float32),
                jax.ShapeDtypeStruct((CR, K), jnp.int32),
            ],
        )(candv, candc)

    parts = {}
    for c in reversed(range(NCH)):
        base = c * CR
        scores_c, thr_c = stage1(base)
        candv_c, candc_c = sc_chunk(base)(scores_c, thr_c)
        parts[c] = stage3(base, candv_c, candc_c)
    out_v = jnp.concatenate([parts[c][0] for c in range(NCH)], axis=0)
    out_i = jnp.concatenate([parts[c][1] for c in range(NCH)], axis=0)
    return out_v, out_i
